# baseline (speedup 1.0000x reference)
import functools

import jax
import jax.numpy as jnp
from jax import lax
from jax.experimental import pallas as pl
from jax.experimental.pallas import tpu as pltpu
from jax.experimental.pallas import tpu_sc as plsc

DIM = 64
_info = plsc.get_sparse_core_info()
NC, NS = _info.num_cores, _info.num_subcores
NW = NC * NS

CHUNK = 640


def _gather_body(n_chunks, idx_hbm, table_hbm, out_hbm,
                 idx_all, rows_v, sem_g, sem_s):
  wid = lax.axis_index("s") * NC + lax.axis_index("c")
  per_w = n_chunks * CHUNK
  base = wid * per_w

  pltpu.sync_copy(idx_hbm.at[pl.ds(base, per_w)], idx_all)

  def start_gather(c, slot):
    pltpu.make_async_copy(
        table_hbm.at[idx_all.at[pl.ds(c * CHUNK, CHUNK)]],
        rows_v.at[slot], sem_g.at[slot]).start()

  def wait_gather(slot):
    pltpu.make_async_copy(
        table_hbm.at[idx_all.at[pl.ds(0, CHUNK)]],
        rows_v.at[slot], sem_g.at[slot]).wait()

  def start_store(c, slot):
    pltpu.make_async_copy(
        rows_v.at[slot],
        out_hbm.at[pl.ds(base + c * CHUNK, CHUNK)], sem_s.at[slot]).start()

  def wait_store(slot):
    pltpu.make_async_copy(
        rows_v.at[slot],
        out_hbm.at[pl.ds(base, CHUNK)], sem_s.at[slot]).wait()

  start_gather(0, 0)

  def body(p, _):
    c0 = 2 * p
    wait_gather(0)

    @pl.when(p > 0)
    def _():
      wait_store(1)

    start_gather(c0 + 1, 1)
    start_store(c0, 0)

    wait_gather(1)
    wait_store(0)

    @pl.when(c0 + 2 < n_chunks)
    def _():
      start_gather(c0 + 2, 0)

    start_store(c0 + 1, 1)
    return 0

  lax.fori_loop(0, n_chunks // 2, body, 0, unroll=False)
  wait_store(1)


def kernel(indices, table):
  batch, seq = indices.shape
  n = batch * seq
  assert n % NW == 0
  per_w = n // NW
  assert per_w % CHUNK == 0
  n_chunks = per_w // CHUNK
  assert n_chunks % 2 == 0

  idx_flat = indices.reshape(n).astype(jnp.int32)

  mesh = plsc.VectorSubcoreMesh(core_axis_name="c", subcore_axis_name="s")
  k = functools.partial(
      pl.kernel,
      mesh=mesh,
      out_type=jax.ShapeDtypeStruct((n, DIM), jnp.float32),
      scratch_types=[
          pltpu.VMEM((per_w,), jnp.int32),
          pltpu.VMEM((2, CHUNK, DIM), jnp.float32),
          pltpu.SemaphoreType.DMA((2,)),
          pltpu.SemaphoreType.DMA((2,)),
      ],
      compiler_params=pltpu.CompilerParams(use_tc_tiling_on_sc=False),
  )(functools.partial(_gather_body, n_chunks))

  out = k(idx_flat, table)
  return out.reshape(batch, seq, DIM)
